# Initial kernel scaffold; baseline (speedup 1.0000x reference)
import jax
import jax.numpy as jnp
from jax.experimental import pallas as pl


def kernel(coords, hidden_state, cell_state, W_ih, W_hh, b_ih, b_hh):
    raise NotImplementedError("write your pallas kernel here")


# speedup vs baseline: 3.6485x; 3.6485x over previous
import jax
import jax.numpy as jnp
from jax.experimental import pallas as pl
from jax.experimental.pallas import tpu as pltpu

N = 100000
H = 128
G = 32
NB = G * G
R = 2000
NBLK = N // R


def _sigmoid(x):
    return 1.0 / (1.0 + jnp.exp(-x))


def _grid_id(coords):
    x = jnp.clip(coords[:, 0], 0.0, 1.0)
    y = jnp.clip(coords[:, 1], 0.0, 1.0)
    ix = jnp.clip(jnp.floor(x * G).astype(jnp.int32), 0, G - 1)
    iy = jnp.clip(jnp.floor(y * G).astype(jnp.int32), 0, G - 1)
    return ix * G + iy


def _lstm_hist_kernel(coords_ref, hid_ref, cell_ref, wih_ref, whh_ref, b_ref,
                      cnew_ref, bins_ref):
    i = pl.program_id(0)
    coords = coords_ref[...]
    x = coords[:, 0:1]
    y = coords[:, 1:2]
    gates = (jnp.dot(hid_ref[...], whh_ref[...], preferred_element_type=jnp.float32)
             + x * wih_ref[0:1, :] + y * wih_ref[1:2, :] + b_ref[...])
    ii = _sigmoid(gates[:, :H])
    ff = _sigmoid(gates[:, H:2 * H])
    gg = jnp.tanh(gates[:, 2 * H:3 * H])
    oo = _sigmoid(gates[:, 3 * H:])
    c_new = ff * cell_ref[...] + ii * gg
    h_new = oo * jnp.tanh(c_new)
    cnew_ref[...] = c_new

    gid = _grid_id(coords)
    oh_t = (jax.lax.broadcasted_iota(jnp.int32, (NB, R), 0)
            == gid[None, :]).astype(jnp.float32)
    contrib = jnp.dot(oh_t, h_new, preferred_element_type=jnp.float32)

    @pl.when(i == 0)
    def _():
        bins_ref[...] = jnp.zeros_like(bins_ref)

    bins_ref[...] += contrib


def _gather_kernel(coords_ref, bins_ref, out_ref):
    gid = _grid_id(coords_ref[...])
    oh = (gid[:, None]
          == jax.lax.broadcasted_iota(jnp.int32, (R, NB), 1)).astype(jnp.float32)
    out_ref[...] = jnp.dot(oh, bins_ref[...], preferred_element_type=jnp.float32)


def kernel(coords, hidden_state, cell_state, W_ih, W_hh, b_ih, b_hh):
    wih = W_ih.T
    whh = W_hh.T
    b = (b_ih + b_hh)[None, :]

    c_new, bins = pl.pallas_call(
        _lstm_hist_kernel,
        grid=(NBLK,),
        in_specs=[
            pl.BlockSpec((R, 2), lambda i: (i, 0)),
            pl.BlockSpec((R, H), lambda i: (i, 0)),
            pl.BlockSpec((R, H), lambda i: (i, 0)),
            pl.BlockSpec((2, 4 * H), lambda i: (0, 0)),
            pl.BlockSpec((H, 4 * H), lambda i: (0, 0)),
            pl.BlockSpec((1, 4 * H), lambda i: (0, 0)),
        ],
        out_specs=[
            pl.BlockSpec((R, H), lambda i: (i, 0)),
            pl.BlockSpec((NB, H), lambda i: (0, 0)),
        ],
        out_shape=[
            jax.ShapeDtypeStruct((N, H), jnp.float32),
            jax.ShapeDtypeStruct((NB, H), jnp.float32),
        ],
    )(coords, hidden_state, cell_state, wih, whh, b)

    h_social = pl.pallas_call(
        _gather_kernel,
        grid=(NBLK,),
        in_specs=[
            pl.BlockSpec((R, 2), lambda i: (i, 0)),
            pl.BlockSpec((NB, H), lambda i: (0, 0)),
        ],
        out_specs=pl.BlockSpec((R, H), lambda i: (i, 0)),
        out_shape=jax.ShapeDtypeStruct((N, H), jnp.float32),
    )(coords, bins)

    return (h_social, c_new)
